# baseline (speedup 1.0000x reference)
import functools
import jax
import jax.numpy as jnp
from jax import lax
from jax.experimental import pallas as pl
from jax.experimental.pallas import tpu as pltpu
from jax.experimental.pallas import tpu_sc as plsc

N = 10000
E = 320000
D = 128
H = 128
C = 40

NC = 2
NS = 16
NW = NC * NS
CH = 125
NCH = 80
GB = 2
DB = 4
NPAD = 10112
ZROWS = NPAD // NS
EPT = NCH * CH
EPAD = EPT * NW
NPD = 10240
DROWS = NPD // NS


def _deg_body(dstm_hbm, out_hbm, dst_v, ones_v, zero_v, acc_sh):
    c = lax.axis_index("c")
    s = lax.axis_index("s")
    wid = s * NC + c

    def _fill(i, _):
        ones_v[pl.ds(pl.multiple_of(i * 16, 16), 16)] = jnp.ones((16,), jnp.float32)
        return 0

    lax.fori_loop(0, 128 // 16, _fill, 0)

    def _zfill(i, _):
        zero_v[pl.ds(pl.multiple_of(i * 16, 16), 16)] = jnp.zeros((16,), jnp.float32)
        return 0

    lax.fori_loop(0, DROWS // 16, _zfill, 0)

    pltpu.sync_copy(zero_v, acc_sh.at[pl.ds(s * DROWS, DROWS)])
    pltpu.sync_copy(dstm_hbm.at[wid], dst_v)
    plsc.subcore_barrier()

    def _chunk(j, _):
        pltpu.sync_copy(ones_v.at[pl.ds(0, CH)], acc_sh.at[dst_v.at[j]], add=True)
        return 0

    lax.fori_loop(0, NCH, _chunk, 0)
    plsc.subcore_barrier()
    pltpu.sync_copy(acc_sh.at[pl.ds(s * DROWS, DROWS)],
                    out_hbm.at[c, pl.ds(s * DROWS, DROWS)])


def _agg_body(g_hbm, srcm_hbm, dstm_hbm, out_hbm,
              src_v, dstr_v, rows_v, zero_v, gs0, gs1, is0, is1, is2, is3,
              acc_sh):
    c = lax.axis_index("c")
    s = lax.axis_index("s")
    wid = s * NC + c
    gsems = (gs0, gs1)
    isems = (is0, is1, is2, is3)

    def _zfill(i, _):
        for j in range(H // 16):
            zero_v[i, pl.ds(j * 16, 16)] = jnp.zeros((16,), jnp.float32)
        return 0

    lax.fori_loop(0, 48, _zfill, 0)
    zcopies = []
    for k in range(13):
        zcopies.append(pltpu.async_copy(
            zero_v, acc_sh.at[pl.ds(s * ZROWS + k * 48, 48), :], gs0))
    zcopies.append(pltpu.async_copy(
        zero_v.at[pl.ds(0, 8), :],
        acc_sh.at[pl.ds(s * ZROWS + 624, 8), :], gs0))
    pltpu.sync_copy(srcm_hbm.at[wid], src_v)
    for d in zcopies:
        d.wait()
    plsc.subcore_barrier()

    def _start_dst(db, j):
        pltpu.async_copy(dstm_hbm.at[wid, j], dstr_v.at[db], isems[db])

    def _wait_dst(db):
        pltpu.make_async_copy(dstm_hbm.at[0, 0], dstr_v.at[db],
                              isems[db]).wait()

    def _start_g(b, j):
        pltpu.async_copy(g_hbm.at[src_v.at[j]], rows_v.at[b], gsems[b])

    def _wait_g(b):
        pltpu.make_async_copy(g_hbm.at[src_v.at[0]], rows_v.at[b],
                              gsems[b]).wait()

    for db in range(DB):
        _start_dst(db, db)
    for b in range(GB):
        _start_g(b, b)

    def _group(g, _):
        for u in range(DB):
            j = g * DB + u
            b = u % GB
            _wait_g(b)
            _wait_dst(u)
            pltpu.sync_copy(rows_v.at[b], acc_sh.at[dstr_v.at[u]], add=True)

            @pl.when(j + GB < NCH)
            def _():
                _start_g(b, j + GB)

            @pl.when(j + DB < NCH)
            def _():
                _start_dst(u, j + DB)

        return 0

    lax.fori_loop(0, NCH // DB, _group, 0)
    plsc.subcore_barrier()
    pltpu.sync_copy(acc_sh.at[pl.ds(s * ZROWS, ZROWS), :],
                    out_hbm.at[c, pl.ds(s * ZROWS, ZROWS), :])


@functools.cache
def _sc_kernels():
    mesh = plsc.VectorSubcoreMesh(
        core_axis_name="c", subcore_axis_name="s",
        num_cores=NC, num_subcores=NS)
    deg_k = pl.kernel(
        _deg_body,
        out_type=jax.ShapeDtypeStruct((NC, NPD), jnp.float32),
        mesh=mesh,
        scratch_types=[
            pltpu.VMEM((NCH, CH), jnp.int32),
            pltpu.VMEM((128,), jnp.float32),
            pltpu.VMEM((DROWS,), jnp.float32),
            pltpu.VMEM_SHARED((NPD,), jnp.float32),
        ],
    )
    agg_k = pl.kernel(
        _agg_body,
        out_type=jax.ShapeDtypeStruct((NC, NPAD, H), jnp.float32),
        mesh=mesh,
        scratch_types=[
            pltpu.VMEM((NCH, CH), jnp.int32),
            pltpu.VMEM((DB, CH), jnp.int32),
            pltpu.VMEM((GB, CH, H), jnp.float32),
            pltpu.VMEM((48, H), jnp.float32),
            pltpu.SemaphoreType.DMA,
            pltpu.SemaphoreType.DMA,
            pltpu.SemaphoreType.DMA,
            pltpu.SemaphoreType.DMA,
            pltpu.SemaphoreType.DMA,
            pltpu.SemaphoreType.DMA,
            pltpu.VMEM_SHARED((NPAD, H), jnp.float32),
        ],
    )
    return deg_k, agg_k


_R = 10000
_G = N // _R


def _mm1_body(x_ref, w_ref, dinv_ref, o_ref):
    g = jnp.dot(x_ref[...], w_ref[...], preferred_element_type=jnp.float32)
    o_ref[...] = g * dinv_ref[...]


_mm1 = pl.pallas_call(
    _mm1_body,
    grid=(_G,),
    in_specs=[
        pl.BlockSpec((_R, D), lambda i: (i, 0)),
        pl.BlockSpec((D, H), lambda i: (0, 0)),
        pl.BlockSpec((_R, 1), lambda i: (i, 0)),
    ],
    out_specs=pl.BlockSpec((_R, H), lambda i: (i, 0)),
    out_shape=jax.ShapeDtypeStruct((N, H), jnp.float32),
)


def _mid_body(acc_ref, gs_ref, b_ref, w_ref, dinv_ref, o_ref):
    a = acc_ref[0] + acc_ref[1] + gs_ref[...]
    h = jnp.maximum(a * dinv_ref[...] + b_ref[0:1, :], 0.0)
    g = jnp.dot(h, w_ref[...], preferred_element_type=jnp.float32)
    o_ref[...] = g * dinv_ref[...]


_mid = pl.pallas_call(
    _mid_body,
    grid=(_G,),
    in_specs=[
        pl.BlockSpec((NC, _R, H), lambda i: (0, i, 0)),
        pl.BlockSpec((_R, H), lambda i: (i, 0)),
        pl.BlockSpec((8, H), lambda i: (0, 0)),
        pl.BlockSpec((H, H), lambda i: (0, 0)),
        pl.BlockSpec((_R, 1), lambda i: (i, 0)),
    ],
    out_specs=pl.BlockSpec((_R, H), lambda i: (i, 0)),
    out_shape=jax.ShapeDtypeStruct((N, H), jnp.float32),
)


def _fin_body(acc_ref, gs_ref, b_ref, w_ref, bl_ref, dinv_ref, o_ref):
    a = acc_ref[0] + acc_ref[1] + gs_ref[...]
    h = jnp.maximum(a * dinv_ref[...] + b_ref[0:1, :], 0.0)
    o_ref[...] = jnp.dot(h, w_ref[...],
                         preferred_element_type=jnp.float32) + bl_ref[0:1, :]


_fin = pl.pallas_call(
    _fin_body,
    grid=(_G,),
    in_specs=[
        pl.BlockSpec((NC, _R, H), lambda i: (0, i, 0)),
        pl.BlockSpec((_R, H), lambda i: (i, 0)),
        pl.BlockSpec((8, H), lambda i: (0, 0)),
        pl.BlockSpec((H, C), lambda i: (0, 0)),
        pl.BlockSpec((8, C), lambda i: (0, 0)),
        pl.BlockSpec((_R, 1), lambda i: (i, 0)),
    ],
    out_specs=pl.BlockSpec((_R, C), lambda i: (i, 0)),
    out_shape=jax.ShapeDtypeStruct((N, C), jnp.float32),
)


def kernel(x, edge_index, W1, b1, W2, b2, W3, b3, Wl, bl):
    src = edge_index[0].astype(jnp.int32).reshape(NW, NCH, CH)
    dst = edge_index[1].astype(jnp.int32).reshape(NW, NCH, CH)

    deg_k, agg_k = _sc_kernels()
    degp = deg_k(dst)
    deg = degp[0, :N] + degp[1, :N] + 1.0
    dinv = lax.rsqrt(jnp.maximum(deg, 1.0)).reshape(N, 1)

    b1t = jnp.tile(b1.reshape(1, H), (8, 1))
    b2t = jnp.tile(b2.reshape(1, H), (8, 1))
    b3t = jnp.tile(b3.reshape(1, H), (8, 1))
    blt = jnp.tile(bl.reshape(1, C), (8, 1))

    gs1 = _mm1(x, W1, dinv)
    acc1 = agg_k(gs1, src, dst)
    gs2 = _mid(acc1, gs1, b1t, W2, dinv)
    acc2 = agg_k(gs2, src, dst)
    gs3 = _mid(acc2, gs2, b2t, W3, dinv)
    acc3 = agg_k(gs3, src, dst)
    return _fin(acc3, gs3, b3t, Wl, blt, dinv)
